# baseline (speedup 1.0000x reference)
import functools

import jax
import jax.numpy as jnp
from jax import lax
from jax.experimental import pallas as pl
from jax.experimental.pallas import tpu as pltpu
from jax.experimental.pallas import tpu_sc as plsc

B, L = 1024, 50
N_CAT, N_CONT = 26, 13
VOCAB, HID, OUT = 100000, 32, 128
T = B * L

SC_CORES = 2
SC_SUBCORES = 16
N_WORKERS = SC_CORES * SC_SUBCORES
TOK_PER_W = T // N_WORKERS


COLS_FULL = VOCAB // 128
TAIL = VOCAB - COLS_FULL * 128
U_FULL = N_CAT * COLS_FULL
TP_ROWS = N_CAT * VOCAB * HID // 128
ROWS_PER_FIELD = VOCAB * HID // 128


def _sc_transpose(tabT):
    mesh = plsc.VectorSubcoreMesh(core_axis_name="c", subcore_axis_name="s")

    @functools.partial(
        pl.kernel,
        out_type=jax.ShapeDtypeStruct((N_CAT * VOCAB, HID), jnp.float32),
        mesh=mesh,
        scratch_types=[
            pltpu.VMEM((HID, 128), jnp.float32),
            pltpu.VMEM((128, HID), jnp.float32),
            pltpu.VMEM((HID, TAIL), jnp.float32),
        ],
        compiler_params=pltpu.CompilerParams(use_tc_tiling_on_sc=False,
                                             needs_layout_passes=False),
    )
    def transpose_kernel(tab_hbm, out_hbm, buf_in, buf_out, buf_tail):
        wid = lax.axis_index("s") * SC_CORES + lax.axis_index("c")
        lanes = jnp.arange(16, dtype=jnp.int32)

        def do_transpose(src, width):
            def prow(p, carry):
                for j in range(8):
                    v = plsc.load_gather(
                        src,
                        [lanes + (j % 2) * 16,
                         jnp.zeros((16,), jnp.int32) + p * 4 + j // 2],
                    )
                    buf_out[p * 4 + j // 2, pl.ds((j % 2) * 16, 16)] = v
                return carry
            lax.fori_loop(0, width * HID // 128, prow, 0)

        n_units = (U_FULL - wid + N_WORKERS - 1) // N_WORKERS

        def unit_body(k, carry):
            u = wid + k * N_WORKERS
            i = u // COLS_FULL
            c = u % COLS_FULL
            pltpu.sync_copy(tab_hbm.at[i, :, pl.ds(c * 128, 128)], buf_in)
            do_transpose(buf_in, 128)
            pltpu.sync_copy(
                buf_out, out_hbm.at[pl.ds(i * VOCAB + c * 128, 128), :]
            )
            return carry

        lax.fori_loop(0, n_units, unit_body, 0)

        @pl.when(wid < N_CAT)
        def _tail():
            i = wid
            pltpu.sync_copy(
                tab_hbm.at[i, :, pl.ds(COLS_FULL * 128, TAIL)], buf_tail
            )
            do_transpose(buf_tail, TAIL)
            pltpu.sync_copy(
                buf_out.at[pl.ds(0, TAIL), :],
                out_hbm.at[pl.ds(i * VOCAB + COLS_FULL * 128, TAIL), :],
            )

    return transpose_kernel(tabT)


def _sc_gather(idx2d, tab_packed):
    mesh = plsc.VectorSubcoreMesh(core_axis_name="c", subcore_axis_name="s")
    NBLK = (N_CAT * HID + 127) // 128

    @functools.partial(
        pl.kernel,
        out_type=jax.ShapeDtypeStruct((NBLK, T, 128), jnp.float32),
        mesh=mesh,
        scratch_types=[
            pltpu.VMEM((TOK_PER_W, HID), jnp.float32),
            pltpu.VMEM((TOK_PER_W, HID), jnp.float32),
            pltpu.VMEM((TOK_PER_W,), jnp.int32),
            pltpu.SemaphoreType.DMA,
            pltpu.SemaphoreType.DMA,
            pltpu.SemaphoreType.DMA,
        ],
        compiler_params=pltpu.CompilerParams(use_tc_tiling_on_sc=False),
    )
    def gather_kernel(idx_hbm, tab_hbm, out_hbm, rows_a, rows_b,
                      idx_v, gsem, wsem_a, wsem_b):
        wid = lax.axis_index("s") * SC_CORES + lax.axis_index("c")
        base = wid * TOK_PER_W

        def out_slice(i):
            return out_hbm.at[
                i // 4, pl.ds(base, TOK_PER_W), pl.ds((i % 4) * HID, HID)
            ]

        bufs = (rows_a, rows_b)
        wsems = (wsem_a, wsem_b)
        for i in range(N_CAT):
            rows, wsem = bufs[i % 2], wsems[i % 2]
            pltpu.sync_copy(idx_hbm.at[i, pl.ds(base, TOK_PER_W)], idx_v)
            if i >= 2:
                pltpu.make_async_copy(rows, out_slice(i - 2), wsem).wait()
            pltpu.async_copy(tab_hbm.at[idx_v], rows, gsem).wait()
            pltpu.async_copy(rows, out_slice(i), wsem)
        for i in (N_CAT - 2, N_CAT - 1):
            pltpu.make_async_copy(bufs[i % 2], out_slice(i), wsems[i % 2]).wait()

    return gather_kernel(idx2d, tab_packed)


def _stats_kernel(cont_ref, stats_ref, s_acc, sq_acc):
    k = pl.program_id(0)
    x = cont_ref[...]
    s = jnp.sum(x, axis=0, keepdims=True)
    sq = jnp.sum(x * x, axis=0, keepdims=True)

    @pl.when(k == 0)
    def _init():
        s_acc[...] = s
        sq_acc[...] = sq

    @pl.when(k > 0)
    def _acc():
        s_acc[...] = s_acc[...] + s
        sq_acc[...] = sq_acc[...] + sq

    @pl.when(k == pl.num_programs(0) - 1)
    def _fin():
        inv_n = 1.0 / T
        mu = s_acc[...] * inv_n
        var = sq_acc[...] * inv_n - mu * mu
        stats_ref[0:1, :] = mu
        stats_ref[1:2, :] = lax.rsqrt(var + 1e-5)


def _fuse_kernel(stats_ref, cont_ref, g_ref, gamma_ref, beta_ref, wc_ref,
                 bc_ref, w3_ref, wout2_ref, bout_ref, out_ref):
    mu = stats_ref[0:1, :]
    rstd = stats_ref[1:2, :]
    xn = (cont_ref[...] - mu) * rstd * gamma_ref[...] + beta_ref[...]
    ce = jnp.dot(xn, wc_ref[...], preferred_element_type=jnp.float32)
    ce = ce + bc_ref[...]
    gv = g_ref[...]
    nblk = gv.shape[0]
    lane = lax.broadcasted_iota(jnp.int32, (nblk, 1, 128), 2)
    blk = lax.broadcasted_iota(jnp.int32, (nblk, 1, 128), 0)
    gv = jnp.where(blk * 128 + lane < N_CAT * HID, gv, 0.0)
    acc = jnp.dot(gv[0], w3_ref[0], preferred_element_type=jnp.float32)
    for j in range(1, nblk):
        acc = acc + jnp.dot(gv[j], w3_ref[j],
                            preferred_element_type=jnp.float32)
    acc = acc + jnp.dot(ce, wout2_ref[...],
                        preferred_element_type=jnp.float32)
    out_ref[...] = acc + bout_ref[...]


def kernel(cat_features, cont_features, emb_tables, bn_gamma, bn_beta,
           W_cont, b_cont, W_out, b_out):
    idx2d = jnp.transpose(cat_features.reshape(T, N_CAT).astype(jnp.int32))
    idx2d = idx2d + (jnp.arange(N_CAT, dtype=jnp.int32) * VOCAB)[:, None]
    cont2d = cont_features.reshape(T, N_CONT)
    gamma2 = bn_gamma.reshape(1, N_CONT)
    beta2 = bn_beta.reshape(1, N_CONT)
    bc2 = b_cont.reshape(1, HID)
    bout2 = b_out.reshape(1, OUT)

    tabT = jnp.transpose(emb_tables, (0, 2, 1))
    tab_packed = _sc_transpose(tabT)

    g3 = _sc_gather(idx2d, tab_packed)
    NBLK = g3.shape[0]

    TBLK = 1024
    n_blk = T // TBLK
    stats = pl.pallas_call(
        _stats_kernel,
        grid=(n_blk,),
        in_specs=[pl.BlockSpec((TBLK, N_CONT), lambda k: (k, 0))],
        out_specs=pl.BlockSpec((2, N_CONT), lambda k: (0, 0)),
        out_shape=jax.ShapeDtypeStruct((2, N_CONT), jnp.float32),
        scratch_shapes=[
            pltpu.VMEM((1, N_CONT), jnp.float32),
            pltpu.VMEM((1, N_CONT), jnp.float32),
        ],
    )(cont2d)

    w3 = jnp.concatenate(
        [W_out[: N_CAT * HID],
         jnp.zeros((NBLK * 128 - N_CAT * HID, OUT), jnp.float32)], axis=0
    ).reshape(NBLK, 128, OUT)
    out2d = pl.pallas_call(
        _fuse_kernel,
        grid=(n_blk,),
        in_specs=[
            pl.BlockSpec((2, N_CONT), lambda k: (0, 0)),
            pl.BlockSpec((TBLK, N_CONT), lambda k: (k, 0)),
            pl.BlockSpec((NBLK, TBLK, 128), lambda k: (0, k, 0)),
            pl.BlockSpec((1, N_CONT), lambda k: (0, 0)),
            pl.BlockSpec((1, N_CONT), lambda k: (0, 0)),
            pl.BlockSpec((N_CONT, HID), lambda k: (0, 0)),
            pl.BlockSpec((1, HID), lambda k: (0, 0)),
            pl.BlockSpec((NBLK, 128, OUT), lambda k: (0, 0, 0)),
            pl.BlockSpec((HID, OUT), lambda k: (0, 0)),
            pl.BlockSpec((1, OUT), lambda k: (0, 0)),
        ],
        out_specs=pl.BlockSpec((TBLK, OUT), lambda k: (k, 0)),
        out_shape=jax.ShapeDtypeStruct((T, OUT), jnp.float32),
    )(stats, cont2d, g3, gamma2, beta2, W_cont, bc2,
      w3, W_out[N_CAT * HID :], bout2)

    return out2d.reshape(B, L, OUT)


# speedup vs baseline: 2.3037x; 2.3037x over previous
import functools

import jax
import jax.numpy as jnp
from jax import lax
from jax.experimental import pallas as pl
from jax.experimental.pallas import tpu as pltpu
from jax.experimental.pallas import tpu_sc as plsc

B, L = 1024, 50
N_CAT, N_CONT = 26, 13
VOCAB, HID, OUT = 100000, 32, 128
T = B * L

SC_CORES = 2
SC_SUBCORES = 16
N_WORKERS = SC_CORES * SC_SUBCORES
TOK_PER_W = T // N_WORKERS


def _sc_gather(idx2d, tab3d):
    mesh = plsc.VectorSubcoreMesh(core_axis_name="c", subcore_axis_name="s")
    NBLK = (N_CAT * HID + 127) // 128

    @functools.partial(
        pl.kernel,
        out_type=jax.ShapeDtypeStruct((NBLK, T, 128), jnp.float32),
        mesh=mesh,
        scratch_types=[
            pltpu.VMEM((TOK_PER_W, HID), jnp.float32),
            pltpu.VMEM((TOK_PER_W, HID), jnp.float32),
            pltpu.VMEM((TOK_PER_W,), jnp.int32),
            pltpu.SemaphoreType.DMA,
            pltpu.SemaphoreType.DMA,
            pltpu.SemaphoreType.DMA,
        ],
        compiler_params=pltpu.CompilerParams(use_tc_tiling_on_sc=False),
    )
    def gather_kernel(idx_hbm, tab_hbm, out_hbm, rows_a, rows_b,
                      idx_v, gsem, wsem_a, wsem_b):
        wid = lax.axis_index("s") * SC_CORES + lax.axis_index("c")
        base = wid * TOK_PER_W

        def out_slice(i):
            return out_hbm.at[
                i // 4, pl.ds(base, TOK_PER_W), pl.ds((i % 4) * HID, HID)
            ]

        bufs = (rows_a, rows_b)
        wsems = (wsem_a, wsem_b)
        for i in range(N_CAT):
            rows, wsem = bufs[i % 2], wsems[i % 2]
            pltpu.sync_copy(idx_hbm.at[i, pl.ds(base, TOK_PER_W)], idx_v)
            if i >= 2:
                pltpu.make_async_copy(rows, out_slice(i - 2), wsem).wait()
            pltpu.async_copy(tab_hbm.at[i].at[idx_v], rows, gsem).wait()
            pltpu.async_copy(rows, out_slice(i), wsem)
        for i in (N_CAT - 2, N_CAT - 1):
            pltpu.make_async_copy(bufs[i % 2], out_slice(i), wsems[i % 2]).wait()

    return gather_kernel(idx2d, tab3d)


def _stats_kernel(cont_ref, stats_ref, s_acc, sq_acc):
    k = pl.program_id(0)
    x = cont_ref[...]
    s = jnp.sum(x, axis=0, keepdims=True)
    sq = jnp.sum(x * x, axis=0, keepdims=True)

    @pl.when(k == 0)
    def _init():
        s_acc[...] = s
        sq_acc[...] = sq

    @pl.when(k > 0)
    def _acc():
        s_acc[...] = s_acc[...] + s
        sq_acc[...] = sq_acc[...] + sq

    @pl.when(k == pl.num_programs(0) - 1)
    def _fin():
        inv_n = 1.0 / T
        mu = s_acc[...] * inv_n
        var = sq_acc[...] * inv_n - mu * mu
        stats_ref[0:1, :] = mu
        stats_ref[1:2, :] = lax.rsqrt(var + 1e-5)


def _fuse_kernel(stats_ref, cont_ref, g_ref, gamma_ref, beta_ref, wc_ref,
                 bc_ref, w3_ref, wout2_ref, bout_ref, out_ref):
    mu = stats_ref[0:1, :]
    rstd = stats_ref[1:2, :]
    xn = (cont_ref[...] - mu) * rstd * gamma_ref[...] + beta_ref[...]
    ce = jnp.dot(xn, wc_ref[...], preferred_element_type=jnp.float32)
    ce = ce + bc_ref[...]
    gv = g_ref[...]
    nblk = gv.shape[0]
    lane = lax.broadcasted_iota(jnp.int32, (nblk, 1, 128), 2)
    blk = lax.broadcasted_iota(jnp.int32, (nblk, 1, 128), 0)
    gv = jnp.where(blk * 128 + lane < N_CAT * HID, gv, 0.0)
    acc = jnp.dot(gv[0], w3_ref[0], preferred_element_type=jnp.float32)
    for j in range(1, nblk):
        acc = acc + jnp.dot(gv[j], w3_ref[j],
                            preferred_element_type=jnp.float32)
    acc = acc + jnp.dot(ce, wout2_ref[...],
                        preferred_element_type=jnp.float32)
    out_ref[...] = acc + bout_ref[...]


def kernel(cat_features, cont_features, emb_tables, bn_gamma, bn_beta,
           W_cont, b_cont, W_out, b_out):
    idx2d = jnp.transpose(cat_features.reshape(T, N_CAT).astype(jnp.int32))
    cont2d = cont_features.reshape(T, N_CONT)
    gamma2 = bn_gamma.reshape(1, N_CONT)
    beta2 = bn_beta.reshape(1, N_CONT)
    bc2 = b_cont.reshape(1, HID)
    bout2 = b_out.reshape(1, OUT)

    g3 = _sc_gather(idx2d, emb_tables)
    NBLK = g3.shape[0]

    TBLK = 1024
    n_blk = T // TBLK
    stats = pl.pallas_call(
        _stats_kernel,
        grid=(n_blk,),
        in_specs=[pl.BlockSpec((TBLK, N_CONT), lambda k: (k, 0))],
        out_specs=pl.BlockSpec((2, N_CONT), lambda k: (0, 0)),
        out_shape=jax.ShapeDtypeStruct((2, N_CONT), jnp.float32),
        scratch_shapes=[
            pltpu.VMEM((1, N_CONT), jnp.float32),
            pltpu.VMEM((1, N_CONT), jnp.float32),
        ],
    )(cont2d)

    w3 = jnp.concatenate(
        [W_out[: N_CAT * HID],
         jnp.zeros((NBLK * 128 - N_CAT * HID, OUT), jnp.float32)], axis=0
    ).reshape(NBLK, 128, OUT)
    out2d = pl.pallas_call(
        _fuse_kernel,
        grid=(n_blk,),
        in_specs=[
            pl.BlockSpec((2, N_CONT), lambda k: (0, 0)),
            pl.BlockSpec((TBLK, N_CONT), lambda k: (k, 0)),
            pl.BlockSpec((NBLK, TBLK, 128), lambda k: (0, k, 0)),
            pl.BlockSpec((1, N_CONT), lambda k: (0, 0)),
            pl.BlockSpec((1, N_CONT), lambda k: (0, 0)),
            pl.BlockSpec((N_CONT, HID), lambda k: (0, 0)),
            pl.BlockSpec((1, HID), lambda k: (0, 0)),
            pl.BlockSpec((NBLK, 128, OUT), lambda k: (0, 0, 0)),
            pl.BlockSpec((HID, OUT), lambda k: (0, 0)),
            pl.BlockSpec((1, OUT), lambda k: (0, 0)),
        ],
        out_specs=pl.BlockSpec((TBLK, OUT), lambda k: (k, 0)),
        out_shape=jax.ShapeDtypeStruct((T, OUT), jnp.float32),
    )(stats, cont2d, g3, gamma2, beta2, W_cont, bc2,
      w3, W_out[N_CAT * HID :], bout2)

    return out2d.reshape(B, L, OUT)
